# baseline (speedup 1.0000x reference)
import functools

import jax
import jax.numpy as jnp
from jax import lax
from jax.experimental import pallas as pl
from jax.experimental.pallas import tpu as pltpu
from jax.experimental.pallas import tpu_sc as plsc

N_NODES = 10000
NPAD = 10240
IN_DIM = 128
HID = 256
OUT_DIM = 64
E = 320000
CHUNK = 128
NC = 2
NS = 16
EC1 = 163840
EP = 2 * EC1
ROWS_PER_SUB = NPAD // NS
ROW_CHUNKS = ROWS_PER_SUB // CHUNK


def _zero_buf(buf, ncols, nrows=CHUNK):
    def z(r, c):
        for j in range(ncols // 16):
            buf[r, pl.ds(j * 16, 16)] = jnp.zeros((16,), jnp.float32)
        return c
    lax.fori_loop(0, nrows, z, 0)


BATCH = 32
GCH = 64
NBUF = 4
DEPTH = 3


def _make_scatter(edges_per_core, offset_tables=False):
    chunks = edges_per_core // (NS * GCH)
    nb = chunks // BATCH
    mesh = plsc.VectorSubcoreMesh(core_axis_name="c", subcore_axis_name="s")

    @functools.partial(
        pl.kernel,
        out_type=jax.ShapeDtypeStruct((NC, NPAD, 128), jnp.float32),
        mesh=mesh,
        scratch_types=[
            pltpu.VMEM((BATCH, 2, GCH), jnp.int32),
            pltpu.VMEM((BATCH, 2, GCH), jnp.int32),
            [pltpu.VMEM((GCH, 128), jnp.float32)] * NBUF,
            [pltpu.SemaphoreType.DMA] * NBUF,
            pltpu.VMEM_SHARED((NPAD, 128), jnp.float32),
            pltpu.SemaphoreType.DMA,
        ],
    )
    def body(table, ed3d, out, eidx0, eidx1, rows, gs, acc, semi):
        cid = lax.axis_index("c")
        sid = lax.axis_index("s")
        eb = [eidx0, eidx1]

        if offset_tables:
            wrow = sid * chunks
            off = (cid * NPAD).astype(jnp.int32)

            def fix(buf):
                for r in range(BATCH):
                    for j in range(GCH // 16):
                        s = pl.ds(j * 16, 16)
                        buf[r, 0, s] = buf[r, 0, s] + off
        else:
            wrow = (cid * NS + sid) * chunks

            def fix(buf):
                pass

        pltpu.sync_copy(ed3d.at[pl.ds(wrow, BATCH)], eidx0)
        fix(eidx0)
        _zero_buf(rows[0], 128, GCH)
        for k in range(ROWS_PER_SUB // GCH):
            pltpu.sync_copy(rows[0], acc.at[pl.ds(sid * ROWS_PER_SUB + k * GCH, GCH)])
        for j in range(DEPTH):
            pltpu.async_copy(table.at[eidx0.at[j, 0]], rows[j], gs[j])
        plsc.subcore_barrier()

        for b in range(nb):
            cur, nxt = eb[b % 2], eb[(b + 1) % 2]
            if b < nb - 1:
                pltpu.sync_copy(ed3d.at[pl.ds(wrow + (b + 1) * BATCH, BATCH)],
                                 nxt)
                fix(nxt)

            def step(k, c, b=b, cur=cur, nxt=nxt):
                for j in range(NBUF):
                    g = NBUF * k + j
                    pltpu.make_async_copy(table.at[cur.at[0, 0]], rows[j],
                                          gs[j]).wait()
                    pltpu.sync_copy(rows[j], acc.at[cur.at[g, 1]], add=True)
                    jn = (j + DEPTH) % NBUF

                    @pl.when(g + DEPTH < BATCH)
                    def _(g=g, jn=jn):
                        pltpu.async_copy(table.at[cur.at[g + DEPTH, 0]],
                                         rows[jn], gs[jn])
                    if b < nb - 1:
                        @pl.when(g + DEPTH >= BATCH)
                        def _(g=g, jn=jn):
                            pltpu.async_copy(table.at[nxt.at[g + DEPTH - BATCH, 0]],
                                             rows[jn], gs[jn])
                return c
            lax.fori_loop(0, BATCH // NBUF, step, 0)
        plsc.subcore_barrier()
        for k in range(ROW_CHUNKS):
            r0 = sid * ROWS_PER_SUB + k * CHUNK
            pltpu.sync_copy(acc.at[pl.ds(r0, CHUNK)], out.at[cid, pl.ds(r0, CHUNK)])

    return body


_scatter_l1 = _make_scatter(EC1)
_scatter_l2 = _make_scatter(EP, offset_tables=True)

_HCHUNKS = EC1 // (NS * CHUNK)
_hist_mesh = plsc.VectorSubcoreMesh(core_axis_name="c", subcore_axis_name="s")


@functools.partial(
    pl.kernel,
    out_type=jax.ShapeDtypeStruct((NC, NPAD, 16), jnp.float32),
    mesh=_hist_mesh,
    scratch_types=[
        pltpu.VMEM((_HCHUNKS, CHUNK), jnp.int32),
        pltpu.VMEM((CHUNK, 16), jnp.float32),
        pltpu.VMEM_SHARED((NPAD, 16), jnp.float32),
    ],
)
def _hist(dst2d, out, didxs, buf, acc):
    cid = lax.axis_index("c")
    sid = lax.axis_index("s")
    pltpu.sync_copy(dst2d.at[pl.ds((cid * NS + sid) * _HCHUNKS, _HCHUNKS)], didxs)
    _zero_buf(buf, 16)
    for k in range(ROW_CHUNKS):
        pltpu.sync_copy(buf, acc.at[pl.ds(sid * ROWS_PER_SUB + k * CHUNK, CHUNK)])

    def ones(r, c):
        buf[r, pl.ds(0, 16)] = jnp.ones((16,), jnp.float32)
        return c
    lax.fori_loop(0, CHUNK, ones, 0)
    plsc.subcore_barrier()

    def step(g, c):
        pltpu.sync_copy(buf, acc.at[didxs.at[g]], add=True)
        return c
    lax.fori_loop(0, _HCHUNKS, step, 0)
    plsc.subcore_barrier()
    for k in range(ROW_CHUNKS):
        r0 = sid * ROWS_PER_SUB + k * CHUNK
        pltpu.sync_copy(acc.at[pl.ds(r0, CHUNK)], out.at[cid, pl.ds(r0, CHUNK)])


BM = 1024


def _tc_a_body(hist_ref, x_ref, dinv_ref, xs_ref):
    i = pl.program_id(0)
    h = hist_ref[...]
    deg = (jnp.sum(h[0], axis=1, keepdims=True)
           + jnp.sum(h[1], axis=1, keepdims=True) + 1.0)
    d = lax.rsqrt(deg)
    rows = lax.broadcasted_iota(jnp.int32, (BM, 1), 0) + i * BM
    d = jnp.where(rows < N_NODES, d, 0.0)
    dinv_ref[...] = d
    xs_ref[...] = d * x_ref[...]


def _tc_b_body(s1_ref, xs_ref, dinv_ref, w1_ref, b1_ref, xs2_ref):
    d = dinv_ref[...]
    t = d * (s1_ref[0] + s1_ref[1] + xs_ref[...])
    h = jnp.maximum(
        jnp.dot(t, w1_ref[...], preferred_element_type=jnp.float32) + b1_ref[...],
        0.0)
    v = d * h
    xs2_ref[0] = v[:, :128]
    xs2_ref[1] = v[:, 128:]


def _tc_c_body(s2_ref, xs2_ref, dinv_ref, w2_ref, b2_ref, wc_ref, bc_ref, out_ref):
    d = dinv_ref[...]
    t0 = d * (s2_ref[0] + xs2_ref[0])
    t1 = d * (s2_ref[1] + xs2_ref[1])
    m = (jnp.dot(t0, w2_ref[0], preferred_element_type=jnp.float32)
         + jnp.dot(t1, w2_ref[1], preferred_element_type=jnp.float32)
         + b2_ref[...])
    h2 = jnp.maximum(m, 0.0)
    out_ref[...] = (jnp.dot(h2, wc_ref[...], preferred_element_type=jnp.float32)
                    + bc_ref[...])


def _tc_a(hist, x_pad):
    grid = NPAD // BM
    return pl.pallas_call(
        _tc_a_body,
        grid=(grid,),
        in_specs=[
            pl.BlockSpec((NC, BM, 16), lambda i: (0, i, 0)),
            pl.BlockSpec((BM, IN_DIM), lambda i: (i, 0)),
        ],
        out_specs=[
            pl.BlockSpec((BM, 1), lambda i: (i, 0)),
            pl.BlockSpec((BM, IN_DIM), lambda i: (i, 0)),
        ],
        out_shape=[
            jax.ShapeDtypeStruct((NPAD, 1), jnp.float32),
            jax.ShapeDtypeStruct((NPAD, IN_DIM), jnp.float32),
        ],
    )(hist, x_pad)


def _tc_b(s1, xs, dinv, W1, b1):
    grid = NPAD // BM
    return pl.pallas_call(
        _tc_b_body,
        grid=(grid,),
        in_specs=[
            pl.BlockSpec((NC, BM, 128), lambda i: (0, i, 0)),
            pl.BlockSpec((BM, IN_DIM), lambda i: (i, 0)),
            pl.BlockSpec((BM, 1), lambda i: (i, 0)),
            pl.BlockSpec((IN_DIM, HID), lambda i: (0, 0)),
            pl.BlockSpec((1, HID), lambda i: (0, 0)),
        ],
        out_specs=pl.BlockSpec((NC, BM, 128), lambda i: (0, i, 0)),
        out_shape=jax.ShapeDtypeStruct((NC, NPAD, 128), jnp.float32),
    )(s1, xs, dinv, W1, b1)


def _tc_c(s2, xs2, dinv, W2, b2, Wc, bc):
    grid = NPAD // BM
    return pl.pallas_call(
        _tc_c_body,
        grid=(grid,),
        in_specs=[
            pl.BlockSpec((NC, BM, 128), lambda i: (0, i, 0)),
            pl.BlockSpec((NC, BM, 128), lambda i: (0, i, 0)),
            pl.BlockSpec((BM, 1), lambda i: (i, 0)),
            pl.BlockSpec((NC, 128, HID), lambda i: (0, 0, 0)),
            pl.BlockSpec((1, HID), lambda i: (0, 0)),
            pl.BlockSpec((HID, OUT_DIM), lambda i: (0, 0)),
            pl.BlockSpec((1, OUT_DIM), lambda i: (0, 0)),
        ],
        out_specs=pl.BlockSpec((BM, OUT_DIM), lambda i: (i, 0)),
        out_shape=jax.ShapeDtypeStruct((N_NODES, OUT_DIM), jnp.float32),
    )(s2, xs2, dinv, W2, b2, Wc, bc)


def kernel(x, edge_index, W1, b1, W2, b2, Wc, bc):
    src = edge_index[0].astype(jnp.int32)
    dst = edge_index[1].astype(jnp.int32)
    fill = N_NODES + jnp.arange(EP - E, dtype=jnp.int32) % (NPAD - N_NODES)
    srcp = jnp.concatenate([src, fill])
    dstp = jnp.concatenate([dst, fill])
    x_pad = jnp.pad(x, ((0, NPAD - N_NODES), (0, 0)))

    dst2d = dstp.reshape(EP // CHUNK, CHUNK)
    srcg = srcp.reshape(EP // GCH, GCH)
    dstg = dstp.reshape(EP // GCH, GCH)
    ed3d = jnp.stack([srcg, dstg], axis=1)
    hist = _hist(dst2d)
    dinv, xs = _tc_a(hist, x_pad)
    s1 = _scatter_l1(xs, ed3d)
    xs2 = _tc_b(s1, xs, dinv, W1, b1.reshape(1, HID))
    s2 = _scatter_l2(xs2.reshape(2 * NPAD, 128), ed3d)
    return _tc_c(s2, xs2, dinv, W2.reshape(NC, 128, HID), b2.reshape(1, HID),
                 Wc, bc.reshape(1, OUT_DIM))


# speedup vs baseline: 1.0183x; 1.0183x over previous
import functools

import jax
import jax.numpy as jnp
from jax import lax
from jax.experimental import pallas as pl
from jax.experimental.pallas import tpu as pltpu
from jax.experimental.pallas import tpu_sc as plsc

N_NODES = 10000
NPAD = 10240
IN_DIM = 128
HID = 256
OUT_DIM = 64
E = 320000
CHUNK = 128
NC = 2
NS = 16
EC1 = 163840
EP = 2 * EC1
ROWS_PER_SUB = NPAD // NS
ROW_CHUNKS = ROWS_PER_SUB // CHUNK


def _zero_buf(buf, ncols, nrows=CHUNK):
    def z(r, c):
        for j in range(ncols // 16):
            buf[r, pl.ds(j * 16, 16)] = jnp.zeros((16,), jnp.float32)
        return c
    lax.fori_loop(0, nrows, z, 0)


BATCH = 32
GCH = 64
NBUF = 4
DEPTH = 3


def _make_scatter(edges_per_core, offset_tables=False):
    chunks = edges_per_core // (NS * GCH)
    nb = chunks // BATCH
    mesh = plsc.VectorSubcoreMesh(core_axis_name="c", subcore_axis_name="s")

    @functools.partial(
        pl.kernel,
        out_type=jax.ShapeDtypeStruct((NC, NPAD, 128), jnp.float32),
        mesh=mesh,
        scratch_types=[
            pltpu.VMEM((BATCH, 2, GCH), jnp.int32),
            pltpu.VMEM((BATCH, 2, GCH), jnp.int32),
            [pltpu.VMEM((GCH, 128), jnp.float32)] * NBUF,
            [pltpu.SemaphoreType.DMA] * NBUF,
            pltpu.VMEM_SHARED((NPAD, 128), jnp.float32),
            pltpu.SemaphoreType.DMA,
        ],
    )
    def body(table, ed3d, out, eidx0, eidx1, rows, gs, acc, semi):
        cid = lax.axis_index("c")
        sid = lax.axis_index("s")
        eb = [eidx0, eidx1]

        if offset_tables:
            wrow = sid * chunks
            off = (cid * NPAD).astype(jnp.int32)

            def fix(buf):
                for r in range(BATCH):
                    for j in range(GCH // 16):
                        s = pl.ds(j * 16, 16)
                        buf[r, 0, s] = buf[r, 0, s] + off
        else:
            wrow = (cid * NS + sid) * chunks

            def fix(buf):
                pass

        pltpu.sync_copy(ed3d.at[pl.ds(wrow, BATCH)], eidx0)
        fix(eidx0)
        _zero_buf(rows[0], 128, GCH)
        for k in range(ROWS_PER_SUB // GCH):
            pltpu.sync_copy(rows[0], acc.at[pl.ds(sid * ROWS_PER_SUB + k * GCH, GCH)])
        for j in range(DEPTH):
            pltpu.async_copy(table.at[eidx0.at[j, 0]], rows[j], gs[j])
        plsc.subcore_barrier()

        for b in range(nb):
            cur, nxt = eb[b % 2], eb[(b + 1) % 2]
            if b < nb - 1:
                pltpu.async_copy(ed3d.at[pl.ds(wrow + (b + 1) * BATCH, BATCH)],
                                 nxt, semi)

            def step(k, c, b=b, cur=cur, nxt=nxt):
                for j in range(NBUF):
                    g = NBUF * k + j
                    pltpu.make_async_copy(table.at[cur.at[0, 0]], rows[j],
                                          gs[j]).wait()
                    pltpu.sync_copy(rows[j], acc.at[cur.at[g, 1]], add=True)
                    jn = (j + DEPTH) % NBUF

                    @pl.when(g + DEPTH < BATCH)
                    def _(g=g, jn=jn):
                        pltpu.async_copy(table.at[cur.at[g + DEPTH, 0]],
                                         rows[jn], gs[jn])
                    if b < nb - 1:
                        @pl.when(g + DEPTH == BATCH)
                        def _(nxt=nxt):
                            pltpu.make_async_copy(
                                ed3d.at[pl.ds(wrow, BATCH)], nxt, semi).wait()
                            fix(nxt)

                        @pl.when(g + DEPTH >= BATCH)
                        def _(g=g, jn=jn):
                            pltpu.async_copy(table.at[nxt.at[g + DEPTH - BATCH, 0]],
                                             rows[jn], gs[jn])
                return c
            lax.fori_loop(0, BATCH // NBUF, step, 0)
        plsc.subcore_barrier()
        for k in range(ROW_CHUNKS):
            r0 = sid * ROWS_PER_SUB + k * CHUNK
            pltpu.sync_copy(acc.at[pl.ds(r0, CHUNK)], out.at[cid, pl.ds(r0, CHUNK)])

    return body


_scatter_l1 = _make_scatter(EC1)
_scatter_l2 = _make_scatter(EP, offset_tables=True)

_HCHUNKS = EC1 // (NS * CHUNK)
_hist_mesh = plsc.VectorSubcoreMesh(core_axis_name="c", subcore_axis_name="s")


@functools.partial(
    pl.kernel,
    out_type=jax.ShapeDtypeStruct((NC, NPAD, 16), jnp.float32),
    mesh=_hist_mesh,
    scratch_types=[
        pltpu.VMEM((_HCHUNKS, CHUNK), jnp.int32),
        pltpu.VMEM((CHUNK, 16), jnp.float32),
        pltpu.VMEM_SHARED((NPAD, 16), jnp.float32),
    ],
)
def _hist(dst2d, out, didxs, buf, acc):
    cid = lax.axis_index("c")
    sid = lax.axis_index("s")
    pltpu.sync_copy(dst2d.at[pl.ds((cid * NS + sid) * _HCHUNKS, _HCHUNKS)], didxs)
    _zero_buf(buf, 16)
    for k in range(ROW_CHUNKS):
        pltpu.sync_copy(buf, acc.at[pl.ds(sid * ROWS_PER_SUB + k * CHUNK, CHUNK)])

    def ones(r, c):
        buf[r, pl.ds(0, 16)] = jnp.ones((16,), jnp.float32)
        return c
    lax.fori_loop(0, CHUNK, ones, 0)
    plsc.subcore_barrier()

    def step(g, c):
        pltpu.sync_copy(buf, acc.at[didxs.at[g]], add=True)
        return c
    lax.fori_loop(0, _HCHUNKS, step, 0)
    plsc.subcore_barrier()
    for k in range(ROW_CHUNKS):
        r0 = sid * ROWS_PER_SUB + k * CHUNK
        pltpu.sync_copy(acc.at[pl.ds(r0, CHUNK)], out.at[cid, pl.ds(r0, CHUNK)])


BM = 1024


def _tc_a_body(hist_ref, x_ref, dinv_ref, xs_ref):
    i = pl.program_id(0)
    h = hist_ref[...]
    deg = (jnp.sum(h[0], axis=1, keepdims=True)
           + jnp.sum(h[1], axis=1, keepdims=True) + 1.0)
    d = lax.rsqrt(deg)
    rows = lax.broadcasted_iota(jnp.int32, (BM, 1), 0) + i * BM
    d = jnp.where(rows < N_NODES, d, 0.0)
    dinv_ref[...] = d
    xs_ref[...] = d * x_ref[...]


def _tc_b_body(s1_ref, xs_ref, dinv_ref, w1_ref, b1_ref, xs2_ref):
    d = dinv_ref[...]
    t = d * (s1_ref[0] + s1_ref[1] + xs_ref[...])
    h = jnp.maximum(
        jnp.dot(t, w1_ref[...], preferred_element_type=jnp.float32) + b1_ref[...],
        0.0)
    v = d * h
    xs2_ref[0] = v[:, :128]
    xs2_ref[1] = v[:, 128:]


def _tc_c_body(s2_ref, xs2_ref, dinv_ref, w2_ref, b2_ref, wc_ref, bc_ref, out_ref):
    d = dinv_ref[...]
    t0 = d * (s2_ref[0] + xs2_ref[0])
    t1 = d * (s2_ref[1] + xs2_ref[1])
    m = (jnp.dot(t0, w2_ref[0], preferred_element_type=jnp.float32)
         + jnp.dot(t1, w2_ref[1], preferred_element_type=jnp.float32)
         + b2_ref[...])
    h2 = jnp.maximum(m, 0.0)
    out_ref[...] = (jnp.dot(h2, wc_ref[...], preferred_element_type=jnp.float32)
                    + bc_ref[...])


def _tc_a(hist, x_pad):
    grid = NPAD // BM
    return pl.pallas_call(
        _tc_a_body,
        grid=(grid,),
        in_specs=[
            pl.BlockSpec((NC, BM, 16), lambda i: (0, i, 0)),
            pl.BlockSpec((BM, IN_DIM), lambda i: (i, 0)),
        ],
        out_specs=[
            pl.BlockSpec((BM, 1), lambda i: (i, 0)),
            pl.BlockSpec((BM, IN_DIM), lambda i: (i, 0)),
        ],
        out_shape=[
            jax.ShapeDtypeStruct((NPAD, 1), jnp.float32),
            jax.ShapeDtypeStruct((NPAD, IN_DIM), jnp.float32),
        ],
    )(hist, x_pad)


def _tc_b(s1, xs, dinv, W1, b1):
    grid = NPAD // BM
    return pl.pallas_call(
        _tc_b_body,
        grid=(grid,),
        in_specs=[
            pl.BlockSpec((NC, BM, 128), lambda i: (0, i, 0)),
            pl.BlockSpec((BM, IN_DIM), lambda i: (i, 0)),
            pl.BlockSpec((BM, 1), lambda i: (i, 0)),
            pl.BlockSpec((IN_DIM, HID), lambda i: (0, 0)),
            pl.BlockSpec((1, HID), lambda i: (0, 0)),
        ],
        out_specs=pl.BlockSpec((NC, BM, 128), lambda i: (0, i, 0)),
        out_shape=jax.ShapeDtypeStruct((NC, NPAD, 128), jnp.float32),
    )(s1, xs, dinv, W1, b1)


def _tc_c(s2, xs2, dinv, W2, b2, Wc, bc):
    grid = NPAD // BM
    return pl.pallas_call(
        _tc_c_body,
        grid=(grid,),
        in_specs=[
            pl.BlockSpec((NC, BM, 128), lambda i: (0, i, 0)),
            pl.BlockSpec((NC, BM, 128), lambda i: (0, i, 0)),
            pl.BlockSpec((BM, 1), lambda i: (i, 0)),
            pl.BlockSpec((NC, 128, HID), lambda i: (0, 0, 0)),
            pl.BlockSpec((1, HID), lambda i: (0, 0)),
            pl.BlockSpec((HID, OUT_DIM), lambda i: (0, 0)),
            pl.BlockSpec((1, OUT_DIM), lambda i: (0, 0)),
        ],
        out_specs=pl.BlockSpec((BM, OUT_DIM), lambda i: (i, 0)),
        out_shape=jax.ShapeDtypeStruct((N_NODES, OUT_DIM), jnp.float32),
    )(s2, xs2, dinv, W2, b2, Wc, bc)


def kernel(x, edge_index, W1, b1, W2, b2, Wc, bc):
    src = edge_index[0].astype(jnp.int32)
    dst = edge_index[1].astype(jnp.int32)
    fill = N_NODES + jnp.arange(EP - E, dtype=jnp.int32) % (NPAD - N_NODES)
    srcp = jnp.concatenate([src, fill])
    dstp = jnp.concatenate([dst, fill])
    x_pad = jnp.pad(x, ((0, NPAD - N_NODES), (0, 0)))

    dst2d = dstp.reshape(EP // CHUNK, CHUNK)
    srcg = srcp.reshape(EP // GCH, GCH)
    dstg = dstp.reshape(EP // GCH, GCH)
    ed3d = jnp.stack([srcg, dstg], axis=1)
    hist = _hist(dst2d)
    dinv, xs = _tc_a(hist, x_pad)
    s1 = _scatter_l1(xs, ed3d)
    xs2 = _tc_b(s1, xs, dinv, W1, b1.reshape(1, HID))
    s2 = _scatter_l2(xs2.reshape(2 * NPAD, 128), ed3d)
    return _tc_c(s2, xs2, dinv, W2.reshape(NC, 128, HID), b2.reshape(1, HID),
                 Wc, bc.reshape(1, OUT_DIM))


# speedup vs baseline: 1.0300x; 1.0114x over previous
import functools

import jax
import jax.numpy as jnp
from jax import lax
from jax.experimental import pallas as pl
from jax.experimental.pallas import tpu as pltpu
from jax.experimental.pallas import tpu_sc as plsc

N_NODES = 10000
NPAD = 10240
IN_DIM = 128
HID = 256
OUT_DIM = 64
E = 320000
CHUNK = 128
NC = 2
NS = 16
EC1 = 163840
EP = 2 * EC1
ROWS_PER_SUB = NPAD // NS
ROW_CHUNKS = ROWS_PER_SUB // CHUNK


def _zero_buf(buf, ncols, nrows=CHUNK):
    def z(r, c):
        for j in range(ncols // 16):
            buf[r, pl.ds(j * 16, 16)] = jnp.zeros((16,), jnp.float32)
        return c
    lax.fori_loop(0, nrows, z, 0)


BATCH = 32
GCH = 64
NBUF = 4
DEPTH = 3


def _make_scatter(edges_per_core, offset_tables=False):
    chunks = edges_per_core // (NS * GCH)
    nb = chunks // BATCH
    mesh = plsc.VectorSubcoreMesh(core_axis_name="c", subcore_axis_name="s")

    @functools.partial(
        pl.kernel,
        out_type=jax.ShapeDtypeStruct((NC, NPAD, 128), jnp.float32),
        mesh=mesh,
        scratch_types=[
            pltpu.VMEM((BATCH, 2, GCH), jnp.int32),
            pltpu.VMEM((BATCH, 2, GCH), jnp.int32),
            [pltpu.VMEM((GCH, 128), jnp.float32)] * NBUF,
            [pltpu.SemaphoreType.DMA] * NBUF,
            pltpu.VMEM_SHARED((NPAD, 128), jnp.float32),
            pltpu.SemaphoreType.DMA,
        ],
    )
    def body(table, ed3d, out, eidx0, eidx1, rows, gs, acc, semi):
        cid = lax.axis_index("c")
        sid = lax.axis_index("s")
        eb = [eidx0, eidx1]

        if offset_tables:
            wrow = sid * chunks
            off = (cid * NPAD).astype(jnp.int32)

            def fix(buf):
                for r in range(BATCH):
                    for j in range(GCH // 16):
                        s = pl.ds(j * 16, 16)
                        buf[r, 0, s] = buf[r, 0, s] + off
        else:
            wrow = (cid * NS + sid) * chunks

            def fix(buf):
                pass

        pltpu.sync_copy(ed3d.at[pl.ds(wrow, BATCH)], eidx0)
        fix(eidx0)
        _zero_buf(rows[0], 128, GCH)
        for k in range(ROWS_PER_SUB // GCH):
            pltpu.sync_copy(rows[0], acc.at[pl.ds(sid * ROWS_PER_SUB + k * GCH, GCH)])
        for j in range(DEPTH):
            pltpu.async_copy(table.at[eidx0.at[j, 0]], rows[j], gs[j])
        plsc.subcore_barrier()

        for b in range(nb):
            cur, nxt = eb[b % 2], eb[(b + 1) % 2]
            if b < nb - 1:
                pltpu.async_copy(ed3d.at[pl.ds(wrow + (b + 1) * BATCH, BATCH)],
                                 nxt, semi)

            def step(k, c, b=b, cur=cur, nxt=nxt):
                for j in range(NBUF):
                    g = NBUF * k + j
                    pltpu.make_async_copy(table.at[cur.at[0, 0]], rows[j],
                                          gs[j]).wait()
                    pltpu.sync_copy(rows[j], acc.at[cur.at[g, 1]], add=True)
                    jn = (j + DEPTH) % NBUF

                    @pl.when(g + DEPTH < BATCH)
                    def _(g=g, jn=jn):
                        pltpu.async_copy(table.at[cur.at[g + DEPTH, 0]],
                                         rows[jn], gs[jn])
                    if b < nb - 1:
                        @pl.when(g + DEPTH == BATCH)
                        def _(nxt=nxt):
                            pltpu.make_async_copy(
                                ed3d.at[pl.ds(wrow, BATCH)], nxt, semi).wait()
                            fix(nxt)

                        @pl.when(g + DEPTH >= BATCH)
                        def _(g=g, jn=jn):
                            pltpu.async_copy(table.at[nxt.at[g + DEPTH - BATCH, 0]],
                                             rows[jn], gs[jn])
                return c
            lax.fori_loop(0, BATCH // NBUF, step, 0)
        plsc.subcore_barrier()
        for k in range(ROW_CHUNKS):
            r0 = sid * ROWS_PER_SUB + k * CHUNK
            pltpu.sync_copy(acc.at[pl.ds(r0, CHUNK)], out.at[cid, pl.ds(r0, CHUNK)])

    return body


_scatter_l1 = _make_scatter(EC1)
_scatter_l2 = _make_scatter(EP, offset_tables=True)

_HCHUNKS = EC1 // (NS * CHUNK)
_hist_mesh = plsc.VectorSubcoreMesh(core_axis_name="c", subcore_axis_name="s")


@functools.partial(
    pl.kernel,
    out_type=jax.ShapeDtypeStruct((NC, NPAD, 16), jnp.float32),
    mesh=_hist_mesh,
    scratch_types=[
        pltpu.VMEM((_HCHUNKS, CHUNK), jnp.int32),
        pltpu.VMEM((CHUNK, 16), jnp.float32),
        pltpu.VMEM_SHARED((NPAD, 16), jnp.float32),
    ],
)
def _hist(dst2d, out, didxs, buf, acc):
    cid = lax.axis_index("c")
    sid = lax.axis_index("s")
    pltpu.sync_copy(dst2d.at[pl.ds((cid * NS + sid) * _HCHUNKS, _HCHUNKS)], didxs)
    _zero_buf(buf, 16)
    for k in range(ROW_CHUNKS):
        pltpu.sync_copy(buf, acc.at[pl.ds(sid * ROWS_PER_SUB + k * CHUNK, CHUNK)])

    def ones(r, c):
        buf[r, pl.ds(0, 16)] = jnp.ones((16,), jnp.float32)
        return c
    lax.fori_loop(0, CHUNK, ones, 0)
    plsc.subcore_barrier()

    def step(g, c):
        pltpu.sync_copy(buf, acc.at[didxs.at[g]], add=True)
        return c
    lax.fori_loop(0, _HCHUNKS, step, 0)
    plsc.subcore_barrier()
    for k in range(ROW_CHUNKS):
        r0 = sid * ROWS_PER_SUB + k * CHUNK
        pltpu.sync_copy(acc.at[pl.ds(r0, CHUNK)], out.at[cid, pl.ds(r0, CHUNK)])


BM = 2048


def _tc_a_body(hist_ref, x_ref, dinv_ref, xs_ref):
    i = pl.program_id(0)
    h = hist_ref[...]
    deg = (jnp.sum(h[0], axis=1, keepdims=True)
           + jnp.sum(h[1], axis=1, keepdims=True) + 1.0)
    d = lax.rsqrt(deg)
    rows = lax.broadcasted_iota(jnp.int32, (BM, 1), 0) + i * BM
    d = jnp.where(rows < N_NODES, d, 0.0)
    dinv_ref[...] = d
    xs_ref[...] = d * x_ref[...]


def _tc_b_body(s1_ref, xs_ref, dinv_ref, w1_ref, b1_ref, xs2_ref):
    d = dinv_ref[...]
    t = d * (s1_ref[0] + s1_ref[1] + xs_ref[...])
    h = jnp.maximum(
        jnp.dot(t, w1_ref[...], preferred_element_type=jnp.float32) + b1_ref[...],
        0.0)
    v = d * h
    xs2_ref[0] = v[:, :128]
    xs2_ref[1] = v[:, 128:]


def _tc_c_body(s2_ref, xs2_ref, dinv_ref, w2_ref, b2_ref, wc_ref, bc_ref, out_ref):
    d = dinv_ref[...]
    t0 = d * (s2_ref[0] + xs2_ref[0])
    t1 = d * (s2_ref[1] + xs2_ref[1])
    m = (jnp.dot(t0, w2_ref[0], preferred_element_type=jnp.float32)
         + jnp.dot(t1, w2_ref[1], preferred_element_type=jnp.float32)
         + b2_ref[...])
    h2 = jnp.maximum(m, 0.0)
    out_ref[...] = (jnp.dot(h2, wc_ref[...], preferred_element_type=jnp.float32)
                    + bc_ref[...])


def _tc_a(hist, x_pad):
    grid = NPAD // BM
    return pl.pallas_call(
        _tc_a_body,
        grid=(grid,),
        in_specs=[
            pl.BlockSpec((NC, BM, 16), lambda i: (0, i, 0)),
            pl.BlockSpec((BM, IN_DIM), lambda i: (i, 0)),
        ],
        out_specs=[
            pl.BlockSpec((BM, 1), lambda i: (i, 0)),
            pl.BlockSpec((BM, IN_DIM), lambda i: (i, 0)),
        ],
        out_shape=[
            jax.ShapeDtypeStruct((NPAD, 1), jnp.float32),
            jax.ShapeDtypeStruct((NPAD, IN_DIM), jnp.float32),
        ],
    )(hist, x_pad)


def _tc_b(s1, xs, dinv, W1, b1):
    grid = NPAD // BM
    return pl.pallas_call(
        _tc_b_body,
        grid=(grid,),
        in_specs=[
            pl.BlockSpec((NC, BM, 128), lambda i: (0, i, 0)),
            pl.BlockSpec((BM, IN_DIM), lambda i: (i, 0)),
            pl.BlockSpec((BM, 1), lambda i: (i, 0)),
            pl.BlockSpec((IN_DIM, HID), lambda i: (0, 0)),
            pl.BlockSpec((1, HID), lambda i: (0, 0)),
        ],
        out_specs=pl.BlockSpec((NC, BM, 128), lambda i: (0, i, 0)),
        out_shape=jax.ShapeDtypeStruct((NC, NPAD, 128), jnp.float32),
    )(s1, xs, dinv, W1, b1)


def _tc_c(s2, xs2, dinv, W2, b2, Wc, bc):
    grid = NPAD // BM
    return pl.pallas_call(
        _tc_c_body,
        grid=(grid,),
        in_specs=[
            pl.BlockSpec((NC, BM, 128), lambda i: (0, i, 0)),
            pl.BlockSpec((NC, BM, 128), lambda i: (0, i, 0)),
            pl.BlockSpec((BM, 1), lambda i: (i, 0)),
            pl.BlockSpec((NC, 128, HID), lambda i: (0, 0, 0)),
            pl.BlockSpec((1, HID), lambda i: (0, 0)),
            pl.BlockSpec((HID, OUT_DIM), lambda i: (0, 0)),
            pl.BlockSpec((1, OUT_DIM), lambda i: (0, 0)),
        ],
        out_specs=pl.BlockSpec((BM, OUT_DIM), lambda i: (i, 0)),
        out_shape=jax.ShapeDtypeStruct((N_NODES, OUT_DIM), jnp.float32),
    )(s2, xs2, dinv, W2, b2, Wc, bc)


def kernel(x, edge_index, W1, b1, W2, b2, Wc, bc):
    src = edge_index[0].astype(jnp.int32)
    dst = edge_index[1].astype(jnp.int32)
    fill = N_NODES + jnp.arange(EP - E, dtype=jnp.int32) % (NPAD - N_NODES)
    srcp = jnp.concatenate([src, fill])
    dstp = jnp.concatenate([dst, fill])
    x_pad = jnp.pad(x, ((0, NPAD - N_NODES), (0, 0)))

    dst2d = dstp.reshape(EP // CHUNK, CHUNK)
    srcg = srcp.reshape(EP // GCH, GCH)
    dstg = dstp.reshape(EP // GCH, GCH)
    ed3d = jnp.stack([srcg, dstg], axis=1)
    hist = _hist(dst2d)
    dinv, xs = _tc_a(hist, x_pad)
    s1 = _scatter_l1(xs, ed3d)
    xs2 = _tc_b(s1, xs, dinv, W1, b1.reshape(1, HID))
    s2 = _scatter_l2(xs2.reshape(2 * NPAD, 128), ed3d)
    return _tc_c(s2, xs2, dinv, W2.reshape(NC, 128, HID), b2.reshape(1, HID),
                 Wc, bc.reshape(1, OUT_DIM))
